# Initial kernel scaffold; baseline (speedup 1.0000x reference)
import jax
import jax.numpy as jnp
from jax.experimental import pallas as pl


def kernel(params, labels, head_table):
    raise NotImplementedError("write your pallas kernel here")


# speedup vs baseline: 1.5889x; 1.5889x over previous
import functools

import jax
import jax.numpy as jnp
from jax import lax
from jax.experimental import pallas as pl
from jax.experimental.pallas import tpu as pltpu
from jax.experimental.pallas import tpu_sc as plsc

NUM_HEAD = 100
EMBED_DIM = 256
BATCH = 16384

_info = plsc.get_sparse_core_info()
_NC, _NS = _info.num_cores, _info.num_subcores
_NW = _NC * _NS
_B_PER_W = BATCH // _NW
_CHUNK = 128
_NCHUNK = _B_PER_W // _CHUNK

_mesh = plsc.VectorSubcoreMesh(core_axis_name="c", subcore_axis_name="s")


@functools.partial(
    pl.kernel,
    mesh=_mesh,
    out_type=jax.ShapeDtypeStruct((BATCH, EMBED_DIM), jnp.float32),
    scratch_types=[
        pltpu.VMEM((_NCHUNK, _CHUNK), jnp.int32),
        pltpu.VMEM((_CHUNK, EMBED_DIM), jnp.float32),
        pltpu.VMEM((_CHUNK, EMBED_DIM), jnp.float32),
        pltpu.SemaphoreType.DMA,
        pltpu.SemaphoreType.DMA,
    ],
)
def _gather_kernel(table_hbm, idx_hbm, out_hbm, idx_v, rows0, rows1, sem0, sem1):
    wid = lax.axis_index("s") * _NC + lax.axis_index("c")
    base = wid * _B_PER_W
    pltpu.sync_copy(idx_hbm.at[wid], idx_v)
    bufs = (rows0, rows1)
    sems = (sem0, sem1)
    handles = [None, None]
    handles[0] = pltpu.async_copy(table_hbm.at[idx_v.at[0]], bufs[0], sems[0])
    for c in range(_NCHUNK):
        if c + 1 < _NCHUNK:
            handles[(c + 1) % 2] = pltpu.async_copy(
                table_hbm.at[idx_v.at[c + 1]], bufs[(c + 1) % 2], sems[(c + 1) % 2]
            )
        handles[c % 2].wait()
        pltpu.sync_copy(bufs[c % 2], out_hbm.at[pl.ds(base + c * _CHUNK, _CHUNK)])


def kernel(params, labels, head_table):
    del params
    idx = labels.reshape(_NW, _NCHUNK, _CHUNK)
    out = _gather_kernel(head_table, idx)
    return out.reshape(BATCH, 1, EMBED_DIM)


# speedup vs baseline: 1.6046x; 1.0099x over previous
import functools

import jax
import jax.numpy as jnp
from jax import lax
from jax.experimental import pallas as pl
from jax.experimental.pallas import tpu as pltpu
from jax.experimental.pallas import tpu_sc as plsc

NUM_HEAD = 100
EMBED_DIM = 256
BATCH = 16384

_info = plsc.get_sparse_core_info()
_NC, _NS = _info.num_cores, _info.num_subcores
_NW = _NC * _NS
_B_PER_W = BATCH // _NW
_CHUNK = 128
_NCHUNK = _B_PER_W // _CHUNK
_NBUF = 3

_mesh = plsc.VectorSubcoreMesh(core_axis_name="c", subcore_axis_name="s")


@functools.partial(
    pl.kernel,
    mesh=_mesh,
    out_type=jax.ShapeDtypeStruct((BATCH, EMBED_DIM), jnp.float32),
    scratch_types=[
        pltpu.VMEM((_NCHUNK, _CHUNK), jnp.int32),
    ]
    + [pltpu.VMEM((_CHUNK, EMBED_DIM), jnp.float32)] * _NBUF
    + [pltpu.SemaphoreType.DMA] * (2 * _NBUF),
)
def _gather_kernel(table_hbm, idx_hbm, out_hbm, idx_v, *scratch):
    bufs = scratch[:_NBUF]
    gsems = scratch[_NBUF : 2 * _NBUF]
    wsems = scratch[2 * _NBUF :]
    sid = lax.axis_index("s")
    wid = sid * _NC + lax.axis_index("c")
    base = wid * _B_PER_W

    pltpu.sync_copy(idx_hbm.at[wid], idx_v)

    g = [None] * _NBUF
    w = [None] * _NBUF
    for c in range(min(_NBUF, _NCHUNK)):
        g[c] = pltpu.async_copy(table_hbm.at[idx_v.at[c]], bufs[c], gsems[c])
    for c in range(_NCHUNK):
        b = c % _NBUF
        g[b].wait()
        w[b] = pltpu.async_copy(
            bufs[b], out_hbm.at[pl.ds(base + c * _CHUNK, _CHUNK)], wsems[b]
        )
        if c + _NBUF < _NCHUNK:
            w[b].wait()
            g[b] = pltpu.async_copy(
                table_hbm.at[idx_v.at[c + _NBUF]], bufs[b], gsems[b]
            )
    for c in range(max(0, _NCHUNK - _NBUF), _NCHUNK):
        w[c % _NBUF].wait()


def kernel(params, labels, head_table):
    del params
    idx = labels.reshape(_NW, _NCHUNK, _CHUNK)
    out = _gather_kernel(head_table, idx)
    return out.reshape(BATCH, 1, EMBED_DIM)


# speedup vs baseline: 2.1586x; 1.3453x over previous
import functools

import jax
import jax.numpy as jnp
from jax import lax
from jax.experimental import pallas as pl
from jax.experimental.pallas import tpu as pltpu
from jax.experimental.pallas import tpu_sc as plsc

NUM_HEAD = 100
EMBED_DIM = 256
BATCH = 16384

_info = plsc.get_sparse_core_info()
_NC, _NS = _info.num_cores, _info.num_subcores
_NW = _NC * _NS
_B_PER_W = BATCH // _NW
_CHUNK = 128
_NCHUNK = _B_PER_W // _CHUNK
_NBUF = 3

_mesh = plsc.VectorSubcoreMesh(core_axis_name="c", subcore_axis_name="s")


@functools.partial(
    pl.kernel,
    mesh=_mesh,
    out_type=jax.ShapeDtypeStruct((BATCH, 1, EMBED_DIM), jnp.float32),
    scratch_types=[
        pltpu.VMEM((_NCHUNK, _CHUNK), jnp.int32),
    ]
    + [pltpu.VMEM((_CHUNK, EMBED_DIM), jnp.float32)] * _NBUF
    + [pltpu.SemaphoreType.DMA] * (2 * _NBUF),
)
def _gather_kernel(table_hbm, idx_hbm, out_hbm, idx_v, *scratch):
    bufs = scratch[:_NBUF]
    gsems = scratch[_NBUF : 2 * _NBUF]
    wsems = scratch[2 * _NBUF :]
    sid = lax.axis_index("s")
    wid = sid * _NC + lax.axis_index("c")
    base = wid * _B_PER_W

    pltpu.sync_copy(idx_hbm.at[wid], idx_v)

    g = [None] * _NBUF
    w = [None] * _NBUF
    for c in range(min(_NBUF, _NCHUNK)):
        g[c] = pltpu.async_copy(table_hbm.at[idx_v.at[c]], bufs[c], gsems[c])
    for c in range(_NCHUNK):
        b = c % _NBUF
        g[b].wait()
        w[b] = pltpu.async_copy(
            bufs[b], out_hbm.at[pl.ds(base + c * _CHUNK, _CHUNK), 0], wsems[b]
        )
        if c + _NBUF < _NCHUNK:
            w[b].wait()
            g[b] = pltpu.async_copy(
                table_hbm.at[idx_v.at[c + _NBUF]], bufs[b], gsems[b]
            )
    for c in range(max(0, _NCHUNK - _NBUF), _NCHUNK):
        w[c % _NBUF].wait()


def kernel(params, labels, head_table):
    del params
    idx = labels.reshape(_NW, _NCHUNK, _CHUNK)
    return _gather_kernel(head_table, idx)
